# Initial kernel scaffold; baseline (speedup 1.0000x reference)
import jax
import jax.numpy as jnp
from jax.experimental import pallas as pl


def kernel(x, edge_index, edge_attr, node_ids, emb, W1, b1, W2, b2, Wfc, bfc):
    raise NotImplementedError("write your pallas kernel here")


# speedup vs baseline: 7.2047x; 7.2047x over previous
import functools

import jax
import jax.numpy as jnp
from jax import lax
from jax.experimental import pallas as pl
from jax.experimental.pallas import tpu as pltpu
from jax.experimental.pallas import tpu_sc as plsc

N = 10000
NPAD = 10240
E = 320000
D_FEAT = 128
EMB_DIM = 1024
HID = 128
OUT = 64

NC = 2
NS = 16
NW = NC * NS

_MESH = dict(core_axis_name="c", subcore_axis_name="s", num_cores=NC,
             num_subcores=NS)

EMB_ROWS_PER_W = NPAD // NW
EMB_CHUNK = 64


@functools.partial(
    pl.kernel,
    out_type=jax.ShapeDtypeStruct((NPAD, EMB_DIM), jnp.float32),
    mesh=plsc.VectorSubcoreMesh(**_MESH),
    compiler_params=pltpu.CompilerParams(needs_layout_passes=False),
    scratch_types=[
        pltpu.VMEM((EMB_CHUNK,), jnp.int32),
        pltpu.VMEM((EMB_CHUNK, EMB_DIM), jnp.float32),
        pltpu.SemaphoreType.DMA,
    ],
)
def _emb_gather(table_hbm, ids_hbm, out_hbm, idx_v, rows_v, sem):
    wid = lax.axis_index("s") * NC + lax.axis_index("c")
    base = wid * EMB_ROWS_PER_W

    def body(i, _):
        off = base + i * EMB_CHUNK
        pltpu.sync_copy(ids_hbm.at[pl.ds(off, EMB_CHUNK)], idx_v)
        pltpu.async_copy(table_hbm.at[idx_v], rows_v, sem).wait()
        pltpu.sync_copy(rows_v, out_hbm.at[pl.ds(off, EMB_CHUNK)])
        return 0

    lax.fori_loop(0, EMB_ROWS_PER_W // EMB_CHUNK, body, 0)


EPAD = 323584
DEG_EPT = EPAD // NW
DEG_CHUNK = 2528
NROW = NPAD // 128


@functools.partial(
    pl.kernel,
    out_type=jax.ShapeDtypeStruct((NW, NPAD), jnp.float32),
    mesh=plsc.VectorSubcoreMesh(**_MESH),
    compiler_params=pltpu.CompilerParams(needs_layout_passes=False),
    scratch_types=[
        pltpu.VMEM((NPAD,), jnp.float32),
        pltpu.VMEM((DEG_CHUNK,), jnp.int32),
        pltpu.VMEM((DEG_CHUNK,), jnp.float32),
    ],
)
def _degree(dst_hbm, ew_hbm, out_hbm, degp_v, dst_v, ew_v):
    cid = lax.axis_index("c")
    sid = lax.axis_index("s")
    wid = sid * NC + cid

    def zero_lane(j, _):
        degp_v[pl.ds(j * 16, 16)] = jnp.zeros((16,), jnp.float32)
        return 0

    lax.fori_loop(0, NPAD // 16, zero_lane, 0)

    ebase = wid * DEG_EPT

    def chunk(ci, _):
        off = ebase + ci * DEG_CHUNK
        pltpu.sync_copy(dst_hbm.at[pl.ds(off, DEG_CHUNK)], dst_v)
        pltpu.sync_copy(ew_hbm.at[pl.ds(off, DEG_CHUNK)], ew_v)

        def grp(i, _):
            nid = dst_v[pl.ds(i * 16, 16)]
            val = ew_v[pl.ds(i * 16, 16)]
            plsc.addupdate_scatter(degp_v, [nid], val)
            return 0

        lax.fori_loop(0, DEG_CHUNK // 16, grp, 0)
        return 0

    lax.fori_loop(0, DEG_EPT // DEG_CHUNK, chunk, 0)
    pltpu.sync_copy(degp_v, out_hbm.at[wid])


MP_EPT = EPAD // NW
MP_CHUNK = 128
MP_ROWS_PER_T = NPAD // NS


@functools.partial(
    pl.kernel,
    out_type=jax.ShapeDtypeStruct((NC, NPAD, HID), jnp.float32),
    mesh=plsc.VectorSubcoreMesh(**_MESH),
    compiler_params=pltpu.CompilerParams(needs_layout_passes=False),
    scratch_types=[
        pltpu.VMEM((MP_CHUNK, HID), jnp.float32),
        pltpu.VMEM((MP_CHUNK,), jnp.int32),
        pltpu.VMEM((MP_CHUNK,), jnp.int32),
        pltpu.VMEM((MP_CHUNK,), jnp.float32),
        pltpu.VMEM_SHARED((NPAD, HID), jnp.float32),
        pltpu.SemaphoreType.DMA,
    ],
)
def _msgpass(q_hbm, src_hbm, dst_hbm, ew_hbm, out_hbm,
             rows_v, src_v, dst_v, ew_v, acc_sh, sem):
    cid = lax.axis_index("c")
    sid = lax.axis_index("s")
    wid = sid * NC + cid

    def zero_row(i, _):
        def zero_lane(j, _):
            rows_v[i, pl.ds(j * 16, 16)] = jnp.zeros((16,), jnp.float32)
            return 0
        lax.fori_loop(0, HID // 16, zero_lane, 0)
        return 0

    lax.fori_loop(0, MP_CHUNK, zero_row, 0)

    tbase = sid * MP_ROWS_PER_T

    def zero_acc(i, _):
        pltpu.sync_copy(
            rows_v, acc_sh.at[pl.ds(tbase + i * MP_CHUNK, MP_CHUNK)])
        return 0

    lax.fori_loop(0, MP_ROWS_PER_T // MP_CHUNK, zero_acc, 0)
    plsc.subcore_barrier()

    ebase = wid * MP_EPT

    def chunk(ci, _):
        off = ebase + ci * MP_CHUNK
        pltpu.sync_copy(src_hbm.at[pl.ds(off, MP_CHUNK)], src_v)
        pltpu.sync_copy(dst_hbm.at[pl.ds(off, MP_CHUNK)], dst_v)
        pltpu.sync_copy(ew_hbm.at[pl.ds(off, MP_CHUNK)], ew_v)
        pltpu.async_copy(q_hbm.at[src_v], rows_v, sem).wait()

        def scale(k, _):
            splat = jnp.zeros((16,), jnp.int32) + k
            w16 = plsc.load_gather(ew_v, [splat])
            for j in range(HID // 16):
                sl = pl.ds(j * 16, 16)
                rows_v[k, sl] = rows_v[k, sl] * w16
            return 0

        lax.fori_loop(0, MP_CHUNK, scale, 0)
        pltpu.sync_copy(rows_v, acc_sh.at[dst_v], add=True)
        return 0

    lax.fori_loop(0, MP_EPT // MP_CHUNK, chunk, 0)
    plsc.subcore_barrier()

    def writeback(i, _):
        sl = pl.ds(tbase + i * MP_CHUNK, MP_CHUNK)
        pltpu.sync_copy(acc_sh.at[sl], out_hbm.at[cid, sl])
        return 0

    lax.fori_loop(0, MP_ROWS_PER_T // MP_CHUNK, writeback, 0)


RB = 256
GRID = NPAD // RB


def _prep_body(degp_ref, dinv_ref):
    deg = 1.0 + jnp.sum(degp_ref[...], axis=0)
    dinv_ref[...] = jnp.where(deg > 0, lax.rsqrt(jnp.maximum(deg, 1e-30)),
                              0.0)


def _prep(deg_parts):
    return pl.pallas_call(
        _prep_body,
        out_shape=jax.ShapeDtypeStruct((NROW, 128), jnp.float32),
    )(deg_parts.reshape(NW, NROW, 128))


def _mm1_body(h_ref, w_ref, dinv_ref, q_ref, st_ref):
    p = jnp.dot(h_ref[...], w_ref[...], preferred_element_type=jnp.float32)
    dinv = dinv_ref[...]
    q = p * dinv
    q_ref[...] = q
    st_ref[...] = q * dinv


def _mm1(h0, W1, dinv_col):
    return pl.pallas_call(
        _mm1_body,
        grid=(GRID,),
        in_specs=[
            pl.BlockSpec((RB, D_FEAT + EMB_DIM), lambda i: (i, 0)),
            pl.BlockSpec((D_FEAT + EMB_DIM, HID), lambda i: (0, 0)),
            pl.BlockSpec((RB, HID), lambda i: (i, 0)),
        ],
        out_specs=[
            pl.BlockSpec((RB, HID), lambda i: (i, 0)),
            pl.BlockSpec((RB, HID), lambda i: (i, 0)),
        ],
        out_shape=[
            jax.ShapeDtypeStruct((NPAD, HID), jnp.float32),
            jax.ShapeDtypeStruct((NPAD, HID), jnp.float32),
        ],
    )(h0, W1, dinv_col)


def _mm2_body(acc_ref, st_ref, b_ref, w_ref, dinv_ref, q_ref, st2_ref):
    dinv = dinv_ref[...]
    agg = dinv * (acc_ref[0] + acc_ref[1]) + st_ref[...] + b_ref[...]
    h = jnp.maximum(agg, 0.0)
    p = jnp.dot(h, w_ref[...], preferred_element_type=jnp.float32)
    q = p * dinv
    q_ref[...] = q
    st2_ref[...] = q * dinv


def _mm2(acc, st1, b1, W2, dinv_col):
    return pl.pallas_call(
        _mm2_body,
        grid=(GRID,),
        in_specs=[
            pl.BlockSpec((NC, RB, HID), lambda i: (0, i, 0)),
            pl.BlockSpec((RB, HID), lambda i: (i, 0)),
            pl.BlockSpec((1, HID), lambda i: (0, 0)),
            pl.BlockSpec((HID, HID), lambda i: (0, 0)),
            pl.BlockSpec((RB, HID), lambda i: (i, 0)),
        ],
        out_specs=[
            pl.BlockSpec((RB, HID), lambda i: (i, 0)),
            pl.BlockSpec((RB, HID), lambda i: (i, 0)),
        ],
        out_shape=[
            jax.ShapeDtypeStruct((NPAD, HID), jnp.float32),
            jax.ShapeDtypeStruct((NPAD, HID), jnp.float32),
        ],
    )(acc, st1, b1, W2, dinv_col)


def _mm3_body(acc_ref, st_ref, b_ref, w_ref, bfc_ref, dinv_ref, out_ref):
    dinv = dinv_ref[...]
    agg = dinv * (acc_ref[0] + acc_ref[1]) + st_ref[...] + b_ref[...]
    h = jnp.maximum(agg, 0.0)
    out_ref[...] = jnp.dot(h, w_ref[...],
                           preferred_element_type=jnp.float32) + bfc_ref[...]


def _mm3(acc, st2, b2, Wfc, bfc, dinv_col):
    return pl.pallas_call(
        _mm3_body,
        grid=(GRID,),
        in_specs=[
            pl.BlockSpec((NC, RB, HID), lambda i: (0, i, 0)),
            pl.BlockSpec((RB, HID), lambda i: (i, 0)),
            pl.BlockSpec((1, HID), lambda i: (0, 0)),
            pl.BlockSpec((HID, OUT), lambda i: (0, 0)),
            pl.BlockSpec((1, OUT), lambda i: (0, 0)),
            pl.BlockSpec((RB, HID), lambda i: (i, 0)),
        ],
        out_specs=pl.BlockSpec((RB, OUT), lambda i: (i, 0)),
        out_shape=jax.ShapeDtypeStruct((NPAD, OUT), jnp.float32),
    )(acc, st2, b2, Wfc, bfc, dinv_col)


@jax.jit
def kernel(x, edge_index, edge_attr, node_ids, emb, W1, b1, W2, b2, Wfc, bfc):
    src = edge_index[0]
    dst = edge_index[1]
    epad = EPAD - E
    src_p = jnp.pad(src, (0, epad))
    dst_p = jnp.pad(dst, (0, epad))
    ew_p = jnp.pad(edge_attr, (0, epad))
    ids_p = jnp.pad(node_ids, (0, NPAD - N))
    x_p = jnp.pad(x, ((0, NPAD - N), (0, 0)))

    erows = _emb_gather(emb, ids_p)
    deg_parts = _degree(dst_p, ew_p)
    dinv = _prep(deg_parts)
    dinv_col = jnp.broadcast_to(
        dinv.reshape(NPAD, 1), (NPAD, HID))

    h0 = jnp.concatenate([x_p, erows], axis=1)
    q1, st1 = _mm1(h0, W1, dinv_col)
    acc1 = _msgpass(q1, src_p, dst_p, ew_p)
    q2, st2 = _mm2(acc1, st1, b1.reshape(1, HID), W2, dinv_col)
    acc2 = _msgpass(q2, src_p, dst_p, ew_p)
    out = _mm3(acc2, st2, b2.reshape(1, HID), Wfc, bfc.reshape(1, OUT),
               dinv_col)
    return out[:N]
